# Initial kernel scaffold; baseline (speedup 1.0000x reference)
import jax
import jax.numpy as jnp
from jax.experimental import pallas as pl


def kernel(instances2d_ids, instance3d, semantic3d_label, geometry):
    raise NotImplementedError("write your pallas kernel here")


# speedup vs baseline: 11.0040x; 11.0040x over previous
import functools

import jax
import jax.numpy as jnp
from jax.experimental import pallas as pl
from jax.experimental.pallas import tpu as pltpu

D = H = W = 64
N = D * H * W
NV = 128
NS = 32
NUM_LABELS = 20
BLK = 4096
INF = 1 << 25
THRESH = 1 << 24
RADIUS = 3


def _hist_kernel(inst_ref, sem_ref, geom_ref, hist_ref):
    pi = pl.program_id(0)

    @pl.when(pi == 0)
    def _():
        hist_ref[...] = jnp.zeros_like(hist_ref)

    inst = inst_ref[...]
    sem = sem_ref[...]
    surf = jnp.abs(geom_ref[...]) < 1.0

    iota_v = jax.lax.broadcasted_iota(jnp.int32, (1, NV), 1)
    iota_s = jax.lax.broadcasted_iota(jnp.int32, (1, NS), 1)
    a = (inst == iota_v).astype(jnp.float32)
    b = ((sem == iota_s) & surf).astype(jnp.float32)
    b = jnp.where(iota_s == NS - 1, 1.0, b)
    hist_ref[...] += jax.lax.dot_general(
        a, b, (((0,), (0,)), ((), ())),
        preferred_element_type=jnp.float32)


def _table_kernel(ids_ref, hist_ref, lut_ref, sem_arr_ref):
    ids2d = ids_ref[...]
    hist = hist_ref[...]
    iota_col = jax.lax.broadcasted_iota(jnp.int32, (NV, 1), 0)
    keep = jnp.any(iota_col == ids2d, axis=1, keepdims=True)
    count_all = hist[:, NS - 1:NS]
    present = keep & (count_all > 0.0) & (iota_col >= 1)
    present_f = present.astype(jnp.float32)
    r_i = jax.lax.broadcasted_iota(jnp.int32, (NV, NV), 0)
    c_i = jax.lax.broadcasted_iota(jnp.int32, (NV, NV), 1)
    lt = (c_i < r_i).astype(jnp.float32)
    rank = jax.lax.dot_general(lt, present_f, (((1,), (0,)), ((), ())),
                               preferred_element_type=jnp.float32)
    covered = jnp.sum(jnp.where(keep & (iota_col >= 1), count_all, 0.0))
    present0 = (covered < float(N)).astype(jnp.int32)
    pid = rank.astype(jnp.int32) + 2 + present0
    lut_ref[...] = jnp.where(present, pid, 0)

    iota_s = jax.lax.broadcasted_iota(jnp.int32, (NV, NS), 1)
    hist_m = jnp.where(iota_s < NUM_LABELS, hist, -1.0)
    cnt = jnp.sum(jnp.where(iota_s < NUM_LABELS, hist, 0.0), axis=1,
                  keepdims=True)
    mx = jnp.max(hist_m, axis=1, keepdims=True)
    sel = jnp.min(jnp.where((hist_m == mx) & (iota_s < NUM_LABELS),
                            iota_s, NS), axis=1, keepdims=True)
    do_sem = present & (cnt > 0.0)
    iota_j = jax.lax.broadcasted_iota(jnp.int32, (1, 512), 1)
    m = ((pid == iota_j) & do_sem).astype(jnp.float32)
    sel_f = sel.astype(jnp.float32)
    sem_vals = jax.lax.dot_general(sel_f, m, (((0,), (0,)), ((), ())),
                                   preferred_element_type=jnp.float32)
    sem_vals = jnp.where(iota_j == 1, 10.0, sem_vals)
    sem_vals = jnp.where(iota_j == 2, 11.0, sem_vals)
    sem_arr_ref[...] = sem_vals


def _map_fill_kernel(ids_ref, lut_ref, inst_ref, sem_ref, geom_ref,
                     pan_ref, zpad_ref, rpad_ref, xpad_ref):
    inst = inst_ref[...]
    sem = sem_ref[...]
    surf = jnp.abs(geom_ref[...]) < 1.0

    def body(k, p):
        v = ids_ref[k]
        lv = lut_ref[v]
        return jnp.where((inst == v) & surf, lv, p)

    p = jax.lax.fori_loop(0, 64, body, jnp.zeros_like(inst))
    s_int = surf.astype(jnp.int32)
    p = jnp.where(sem == 0, 1, p)
    p = jnp.where(sem == s_int, 2, p)

    unassigned = surf & (p == 0)
    penc = jnp.where(p == 0, INF, p)

    rows = jax.lax.broadcasted_iota(jnp.int32, (D * H, W), 0)
    hmod = jnp.bitwise_and(rows, H - 1)

    zpad_ref[...] = jnp.full((D * H, 128), INF, jnp.int32)
    zpad_ref[:, RADIUS:RADIUS + W] = penc
    t = jnp.full((D * H, W), INF, jnp.int32)
    for dz in range(-RADIUS, RADIUS):
        s = zpad_ref[:, RADIUS + dz:RADIUS + dz + W] + (dz + RADIUS) * 1024
        t = jnp.minimum(t, s)

    rpad_ref[...] = jnp.full((D * H + 8, W), INF, jnp.int32)
    rpad_ref[RADIUS:RADIUS + D * H, :] = t
    t = jnp.full((D * H, W), INF, jnp.int32)
    for dy in range(-RADIUS, RADIUS):
        s = rpad_ref[RADIUS + dy:RADIUS + dy + D * H, :] + (dy + RADIUS) * 6144
        ok = (hmod + dy >= 0) & (hmod + dy < H)
        t = jnp.minimum(t, jnp.where(ok, s, INF))

    xpad_ref[...] = jnp.full((D * H + 384, W), INF, jnp.int32)
    xpad_ref[RADIUS * H:RADIUS * H + D * H, :] = t
    t = jnp.full((D * H, W), INF, jnp.int32)
    for dx in range(-RADIUS, RADIUS):
        base = RADIUS * H + dx * H
        s = xpad_ref[base:base + D * H, :] + (dx + RADIUS) * 36864
        t = jnp.minimum(t, s)

    fill = jnp.where(t < THRESH, jnp.bitwise_and(t, 1023), 0)
    out = jnp.where(unassigned, fill, p)
    pan_ref[...] = out.astype(jnp.float32)


@functools.partial(jax.jit, static_argnames=("interpret",))
def _run(instances2d_ids, instance3d, semantic3d_label, geometry,
         interpret=False):
    inst_col = instance3d.reshape(N, 1)
    sem_col = semantic3d_label.reshape(N, 1)
    geom_col = geometry.reshape(N, 1)

    hist = pl.pallas_call(
        _hist_kernel,
        grid=(N // BLK,),
        in_specs=[
            pl.BlockSpec((BLK, 1), lambda i: (i, 0)),
            pl.BlockSpec((BLK, 1), lambda i: (i, 0)),
            pl.BlockSpec((BLK, 1), lambda i: (i, 0)),
        ],
        out_specs=pl.BlockSpec((NV, NS), lambda i: (0, 0)),
        out_shape=jax.ShapeDtypeStruct((NV, NS), jnp.float32),
        interpret=interpret,
    )(inst_col, sem_col, geom_col)

    ids2d = (instances2d_ids.astype(jnp.int32) + 1).reshape(1, 64)
    lut, sem_arr = pl.pallas_call(
        _table_kernel,
        in_specs=[pl.BlockSpec(memory_space=pltpu.VMEM),
                  pl.BlockSpec(memory_space=pltpu.VMEM)],
        out_specs=[pl.BlockSpec(memory_space=pltpu.VMEM),
                   pl.BlockSpec(memory_space=pltpu.VMEM)],
        out_shape=[jax.ShapeDtypeStruct((NV, 1), jnp.int32),
                   jax.ShapeDtypeStruct((1, 512), jnp.float32)],
        interpret=interpret,
    )(ids2d, hist)

    inst2 = instance3d.reshape(D * H, W)
    sem2 = semantic3d_label.reshape(D * H, W)
    geom2 = geometry.reshape(D * H, W)
    pan = pl.pallas_call(
        _map_fill_kernel,
        in_specs=[pl.BlockSpec(memory_space=pltpu.SMEM),
                  pl.BlockSpec(memory_space=pltpu.SMEM),
                  pl.BlockSpec(memory_space=pltpu.VMEM),
                  pl.BlockSpec(memory_space=pltpu.VMEM),
                  pl.BlockSpec(memory_space=pltpu.VMEM)],
        out_specs=pl.BlockSpec(memory_space=pltpu.VMEM),
        out_shape=jax.ShapeDtypeStruct((D * H, W), jnp.float32),
        scratch_shapes=[pltpu.VMEM((D * H, 128), jnp.int32),
                        pltpu.VMEM((D * H + 8, W), jnp.int32),
                        pltpu.VMEM((D * H + 384, W), jnp.int32)],
        interpret=interpret,
    )(ids2d.reshape(64), lut.reshape(NV), inst2, sem2, geom2)

    return pan.reshape(1, 1, D, H, W), sem_arr.reshape(512)


def kernel(instances2d_ids, instance3d, semantic3d_label, geometry):
    return _run(instances2d_ids, instance3d, semantic3d_label, geometry)


# speedup vs baseline: 11.3033x; 1.0272x over previous
import functools

import jax
import jax.numpy as jnp
from jax.experimental import pallas as pl
from jax.experimental.pallas import tpu as pltpu

D = H = W = 64
N = D * H * W
NV = 128
NS = 32
NUM_LABELS = 20
BLK = 4096
INF = 1 << 25
THRESH = 1 << 24
RADIUS = 3


def _hist_kernel(inst_ref, sem_ref, geom_ref, hist_ref):
    pi = pl.program_id(0)

    @pl.when(pi == 0)
    def _():
        hist_ref[...] = jnp.zeros_like(hist_ref)

    inst = inst_ref[...]
    sem = sem_ref[...]
    surf = jnp.abs(geom_ref[...]) < 1.0

    iota_v = jax.lax.broadcasted_iota(jnp.int32, (1, NV), 1)
    iota_s = jax.lax.broadcasted_iota(jnp.int32, (1, NS), 1)
    a = (inst == iota_v).astype(jnp.float32)
    b = ((sem == iota_s) & surf).astype(jnp.float32)
    b = jnp.where(iota_s == NS - 1, 1.0, b)
    hist_ref[...] += jax.lax.dot_general(
        a, b, (((0,), (0,)), ((), ())),
        preferred_element_type=jnp.float32)


def _table_kernel(ids_ref, hist_ref, lut_ref, sem_arr_ref):
    ids2d = ids_ref[...]
    hist = hist_ref[...]
    iota_col = jax.lax.broadcasted_iota(jnp.int32, (NV, 1), 0)
    keep = jnp.any(iota_col == ids2d, axis=1, keepdims=True)
    count_all = hist[:, NS - 1:NS]
    present = keep & (count_all > 0.0) & (iota_col >= 1)
    present_f = present.astype(jnp.float32)
    r_i = jax.lax.broadcasted_iota(jnp.int32, (NV, NV), 0)
    c_i = jax.lax.broadcasted_iota(jnp.int32, (NV, NV), 1)
    lt = (c_i < r_i).astype(jnp.float32)
    rank = jax.lax.dot_general(lt, present_f, (((1,), (0,)), ((), ())),
                               preferred_element_type=jnp.float32)
    covered = jnp.sum(jnp.where(keep & (iota_col >= 1), count_all, 0.0))
    present0 = (covered < float(N)).astype(jnp.int32)
    pid = rank.astype(jnp.int32) + 2 + present0
    lut_ref[...] = jnp.where(present, pid, 0)

    iota_s = jax.lax.broadcasted_iota(jnp.int32, (NV, NS), 1)
    hist_m = jnp.where(iota_s < NUM_LABELS, hist, -1.0)
    cnt = jnp.sum(jnp.where(iota_s < NUM_LABELS, hist, 0.0), axis=1,
                  keepdims=True)
    mx = jnp.max(hist_m, axis=1, keepdims=True)
    sel = jnp.min(jnp.where((hist_m == mx) & (iota_s < NUM_LABELS),
                            iota_s, NS), axis=1, keepdims=True)
    do_sem = present & (cnt > 0.0)
    iota_j = jax.lax.broadcasted_iota(jnp.int32, (1, 512), 1)
    m = ((pid == iota_j) & do_sem).astype(jnp.float32)
    sel_f = sel.astype(jnp.float32)
    sem_vals = jax.lax.dot_general(sel_f, m, (((0,), (0,)), ((), ())),
                                   preferred_element_type=jnp.float32)
    sem_vals = jnp.where(iota_j == 1, 10.0, sem_vals)
    sem_vals = jnp.where(iota_j == 2, 11.0, sem_vals)
    sem_arr_ref[...] = sem_vals


def _map_fill_kernel(ids_ref, lut_ref, inst_ref, sem_ref, geom_ref,
                     pan_ref, zpad_ref, ypad_ref, xpad_ref):
    inst = inst_ref[...]
    sem = sem_ref[...]
    surf = jnp.abs(geom_ref[...]) < 1.0

    def body(k, p):
        v = ids_ref[k]
        lv = lut_ref[v]
        return jnp.where((inst == v) & surf, lv, p)

    p = jax.lax.fori_loop(0, 64, body, jnp.zeros_like(inst))
    s_int = surf.astype(jnp.int32)
    p = jnp.where(sem == 0, 1, p)
    p = jnp.where(sem == s_int, 2, p)

    unassigned = surf & (p == 0)
    penc = jnp.where(p == 0, INF, p)

    lanes = jax.lax.broadcasted_iota(jnp.int32, (D, H * W), 1)
    wmod = jnp.bitwise_and(lanes, W - 1)
    hidx = jnp.right_shift(lanes, 6)

    zpad_ref[...] = jnp.full((D, H * W + 256), INF, jnp.int32)
    zpad_ref[:, 128:128 + H * W] = penc
    t = jnp.full((D, H * W), INF, jnp.int32)
    for dz in range(-RADIUS, RADIUS):
        s = zpad_ref[:, 128 + dz:128 + dz + H * W] + (dz + RADIUS) * 1024
        ok = (wmod + dz >= 0) & (wmod + dz < W)
        t = jnp.minimum(t, jnp.where(ok, s, INF))

    ypad_ref[...] = jnp.full((D, H * W + 512), INF, jnp.int32)
    ypad_ref[:, 256:256 + H * W] = t
    t = jnp.full((D, H * W), INF, jnp.int32)
    for dy in range(-RADIUS, RADIUS):
        s = ypad_ref[:, 256 + dy * W:256 + dy * W + H * W] + (dy + RADIUS) * 6144
        ok = (hidx + dy >= 0) & (hidx + dy < H)
        t = jnp.minimum(t, jnp.where(ok, s, INF))

    xpad_ref[...] = jnp.full((D + 8, H * W), INF, jnp.int32)
    xpad_ref[RADIUS:RADIUS + D, :] = t
    t = jnp.full((D, H * W), INF, jnp.int32)
    for dx in range(-RADIUS, RADIUS):
        s = xpad_ref[RADIUS + dx:RADIUS + dx + D, :] + (dx + RADIUS) * 36864
        t = jnp.minimum(t, s)

    fill = jnp.where(t < THRESH, jnp.bitwise_and(t, 1023), 0)
    out = jnp.where(unassigned, fill, p)
    pan_ref[...] = out.astype(jnp.float32)


@functools.partial(jax.jit, static_argnames=("interpret",))
def _run(instances2d_ids, instance3d, semantic3d_label, geometry,
         interpret=False):
    inst_col = instance3d.reshape(N, 1)
    sem_col = semantic3d_label.reshape(N, 1)
    geom_col = geometry.reshape(N, 1)

    hist = pl.pallas_call(
        _hist_kernel,
        grid=(N // BLK,),
        in_specs=[
            pl.BlockSpec((BLK, 1), lambda i: (i, 0)),
            pl.BlockSpec((BLK, 1), lambda i: (i, 0)),
            pl.BlockSpec((BLK, 1), lambda i: (i, 0)),
        ],
        out_specs=pl.BlockSpec((NV, NS), lambda i: (0, 0)),
        out_shape=jax.ShapeDtypeStruct((NV, NS), jnp.float32),
        interpret=interpret,
    )(inst_col, sem_col, geom_col)

    ids2d = (instances2d_ids.astype(jnp.int32) + 1).reshape(1, 64)
    lut, sem_arr = pl.pallas_call(
        _table_kernel,
        in_specs=[pl.BlockSpec(memory_space=pltpu.VMEM),
                  pl.BlockSpec(memory_space=pltpu.VMEM)],
        out_specs=[pl.BlockSpec(memory_space=pltpu.VMEM),
                   pl.BlockSpec(memory_space=pltpu.VMEM)],
        out_shape=[jax.ShapeDtypeStruct((NV, 1), jnp.int32),
                   jax.ShapeDtypeStruct((1, 512), jnp.float32)],
        interpret=interpret,
    )(ids2d, hist)

    inst2 = instance3d.reshape(D, H * W)
    sem2 = semantic3d_label.reshape(D, H * W)
    geom2 = geometry.reshape(D, H * W)
    pan = pl.pallas_call(
        _map_fill_kernel,
        in_specs=[pl.BlockSpec(memory_space=pltpu.SMEM),
                  pl.BlockSpec(memory_space=pltpu.SMEM),
                  pl.BlockSpec(memory_space=pltpu.VMEM),
                  pl.BlockSpec(memory_space=pltpu.VMEM),
                  pl.BlockSpec(memory_space=pltpu.VMEM)],
        out_specs=pl.BlockSpec(memory_space=pltpu.VMEM),
        out_shape=jax.ShapeDtypeStruct((D, H * W), jnp.float32),
        scratch_shapes=[pltpu.VMEM((D, H * W + 256), jnp.int32),
                        pltpu.VMEM((D, H * W + 512), jnp.int32),
                        pltpu.VMEM((D + 8, H * W), jnp.int32)],
        interpret=interpret,
    )(ids2d.reshape(64), lut.reshape(NV), inst2, sem2, geom2)

    return pan.reshape(1, 1, D, H, W), sem_arr.reshape(512)


def kernel(instances2d_ids, instance3d, semantic3d_label, geometry):
    return _run(instances2d_ids, instance3d, semantic3d_label, geometry)


# speedup vs baseline: 21.1690x; 1.8728x over previous
import functools

import jax
import jax.numpy as jnp
from jax.experimental import pallas as pl
from jax.experimental.pallas import tpu as pltpu

D = H = W = 64
N = D * H * W
NV = 128
NS = 32
NUM_LABELS = 20
BLK = 4096
INF = 1 << 25
THRESH = 1 << 24
RADIUS = 3


def _hist_kernel(inst_ref, sem_ref, geom_ref, hist_ref):
    iota_v = jax.lax.broadcasted_iota(jnp.int32, (NV, 1), 0)
    iota_s = jax.lax.broadcasted_iota(jnp.int32, (NS, 1), 0)

    def row(r, acc):
        inst_row = inst_ref[r, :].reshape(1, 128)
        sem_row = sem_ref[r, :].reshape(1, 128)
        surf_row = (jnp.abs(geom_ref[r, :]) < 1.0).reshape(1, 128)
        a = (inst_row == iota_v).astype(jnp.float32)
        b = ((sem_row == iota_s) & surf_row).astype(jnp.float32)
        b = jnp.where(iota_s == NS - 1, 1.0, b)
        return acc + jax.lax.dot_general(
            a, b, (((1,), (1,)), ((), ())),
            preferred_element_type=jnp.float32)

    hist_ref[...] = jax.lax.fori_loop(
        0, N // 128, row, jnp.zeros((NV, NS), jnp.float32))


def _table_kernel(ids_ref, hist_ref, lut_ref, sem_arr_ref):
    ids2d = ids_ref[...]
    hist = hist_ref[...]
    iota_col = jax.lax.broadcasted_iota(jnp.int32, (NV, 1), 0)
    keep = jnp.any(iota_col == ids2d, axis=1, keepdims=True)
    count_all = hist[:, NS - 1:NS]
    present = keep & (count_all > 0.0) & (iota_col >= 1)
    present_f = present.astype(jnp.float32)
    r_i = jax.lax.broadcasted_iota(jnp.int32, (NV, NV), 0)
    c_i = jax.lax.broadcasted_iota(jnp.int32, (NV, NV), 1)
    lt = (c_i < r_i).astype(jnp.float32)
    rank = jax.lax.dot_general(lt, present_f, (((1,), (0,)), ((), ())),
                               preferred_element_type=jnp.float32)
    covered = jnp.sum(jnp.where(keep & (iota_col >= 1), count_all, 0.0))
    present0 = (covered < float(N)).astype(jnp.int32)
    pid = rank.astype(jnp.int32) + 2 + present0
    lut_ref[...] = jnp.where(present, pid, 0)

    iota_s = jax.lax.broadcasted_iota(jnp.int32, (NV, NS), 1)
    hist_m = jnp.where(iota_s < NUM_LABELS, hist, -1.0)
    cnt = jnp.sum(jnp.where(iota_s < NUM_LABELS, hist, 0.0), axis=1,
                  keepdims=True)
    mx = jnp.max(hist_m, axis=1, keepdims=True)
    sel = jnp.min(jnp.where((hist_m == mx) & (iota_s < NUM_LABELS),
                            iota_s, NS), axis=1, keepdims=True)
    do_sem = present & (cnt > 0.0)
    iota_j = jax.lax.broadcasted_iota(jnp.int32, (1, 512), 1)
    m = ((pid == iota_j) & do_sem).astype(jnp.float32)
    sel_f = sel.astype(jnp.float32)
    sem_vals = jax.lax.dot_general(sel_f, m, (((0,), (0,)), ((), ())),
                                   preferred_element_type=jnp.float32)
    sem_vals = jnp.where(iota_j == 1, 10.0, sem_vals)
    sem_vals = jnp.where(iota_j == 2, 11.0, sem_vals)
    sem_arr_ref[...] = sem_vals


def _map_fill_kernel(ids_ref, lut_ref, inst_ref, sem_ref, geom_ref,
                     pan_ref, zpad_ref, ypad_ref, xpad_ref):
    inst = inst_ref[...]
    sem = sem_ref[...]
    surf = jnp.abs(geom_ref[...]) < 1.0

    def body(k, p):
        v = ids_ref[k]
        lv = lut_ref[v]
        return jnp.where((inst == v) & surf, lv, p)

    p = jax.lax.fori_loop(0, 64, body, jnp.zeros_like(inst))
    s_int = surf.astype(jnp.int32)
    p = jnp.where(sem == 0, 1, p)
    p = jnp.where(sem == s_int, 2, p)

    unassigned = surf & (p == 0)
    penc = jnp.where(p == 0, INF, p)

    lanes = jax.lax.broadcasted_iota(jnp.int32, (D, H * W), 1)
    wmod = jnp.bitwise_and(lanes, W - 1)
    hidx = jnp.right_shift(lanes, 6)

    zpad_ref[...] = jnp.full((D, H * W + 256), INF, jnp.int32)
    zpad_ref[:, 128:128 + H * W] = penc
    t = jnp.full((D, H * W), INF, jnp.int32)
    for dz in range(-RADIUS, RADIUS):
        s = zpad_ref[:, 128 + dz:128 + dz + H * W] + (dz + RADIUS) * 1024
        ok = (wmod + dz >= 0) & (wmod + dz < W)
        t = jnp.minimum(t, jnp.where(ok, s, INF))

    ypad_ref[...] = jnp.full((D, H * W + 512), INF, jnp.int32)
    ypad_ref[:, 256:256 + H * W] = t
    t = jnp.full((D, H * W), INF, jnp.int32)
    for dy in range(-RADIUS, RADIUS):
        s = ypad_ref[:, 256 + dy * W:256 + dy * W + H * W] + (dy + RADIUS) * 6144
        ok = (hidx + dy >= 0) & (hidx + dy < H)
        t = jnp.minimum(t, jnp.where(ok, s, INF))

    xpad_ref[...] = jnp.full((D + 8, H * W), INF, jnp.int32)
    xpad_ref[RADIUS:RADIUS + D, :] = t
    t = jnp.full((D, H * W), INF, jnp.int32)
    for dx in range(-RADIUS, RADIUS):
        s = xpad_ref[RADIUS + dx:RADIUS + dx + D, :] + (dx + RADIUS) * 36864
        t = jnp.minimum(t, s)

    fill = jnp.where(t < THRESH, jnp.bitwise_and(t, 1023), 0)
    out = jnp.where(unassigned, fill, p)
    pan_ref[...] = out.astype(jnp.float32)


@functools.partial(jax.jit, static_argnames=("interpret",))
def _run(instances2d_ids, instance3d, semantic3d_label, geometry,
         interpret=False):
    inst_r = instance3d.reshape(N // 128, 128)
    sem_r = semantic3d_label.reshape(N // 128, 128)
    geom_r = geometry.reshape(N // 128, 128)

    hist = pl.pallas_call(
        _hist_kernel,
        in_specs=[pl.BlockSpec(memory_space=pltpu.VMEM)] * 3,
        out_specs=pl.BlockSpec(memory_space=pltpu.VMEM),
        out_shape=jax.ShapeDtypeStruct((NV, NS), jnp.float32),
        interpret=interpret,
    )(inst_r, sem_r, geom_r)

    ids2d = (instances2d_ids.astype(jnp.int32) + 1).reshape(1, 64)
    lut, sem_arr = pl.pallas_call(
        _table_kernel,
        in_specs=[pl.BlockSpec(memory_space=pltpu.VMEM),
                  pl.BlockSpec(memory_space=pltpu.VMEM)],
        out_specs=[pl.BlockSpec(memory_space=pltpu.VMEM),
                   pl.BlockSpec(memory_space=pltpu.VMEM)],
        out_shape=[jax.ShapeDtypeStruct((NV, 1), jnp.int32),
                   jax.ShapeDtypeStruct((1, 512), jnp.float32)],
        interpret=interpret,
    )(ids2d, hist)

    inst2 = instance3d.reshape(D, H * W)
    sem2 = semantic3d_label.reshape(D, H * W)
    geom2 = geometry.reshape(D, H * W)
    pan = pl.pallas_call(
        _map_fill_kernel,
        in_specs=[pl.BlockSpec(memory_space=pltpu.SMEM),
                  pl.BlockSpec(memory_space=pltpu.SMEM),
                  pl.BlockSpec(memory_space=pltpu.VMEM),
                  pl.BlockSpec(memory_space=pltpu.VMEM),
                  pl.BlockSpec(memory_space=pltpu.VMEM)],
        out_specs=pl.BlockSpec(memory_space=pltpu.VMEM),
        out_shape=jax.ShapeDtypeStruct((D, H * W), jnp.float32),
        scratch_shapes=[pltpu.VMEM((D, H * W + 256), jnp.int32),
                        pltpu.VMEM((D, H * W + 512), jnp.int32),
                        pltpu.VMEM((D + 8, H * W), jnp.int32)],
        interpret=interpret,
    )(ids2d.reshape(64), lut.reshape(NV), inst2, sem2, geom2)

    return pan.reshape(1, 1, D, H, W), sem_arr.reshape(512)


def kernel(instances2d_ids, instance3d, semantic3d_label, geometry):
    return _run(instances2d_ids, instance3d, semantic3d_label, geometry)


# speedup vs baseline: 65.3644x; 3.0877x over previous
import functools

import jax
import jax.numpy as jnp
from jax.experimental import pallas as pl
from jax.experimental.pallas import tpu as pltpu

D = H = W = 64
N = D * H * W
NV = 128
NS = 32
NUM_LABELS = 20
BLK = 4096
INF = 1 << 25
THRESH = 1 << 24
RADIUS = 3


def _hist_kernel(inst_ref, sem_ref, geom_ref, hist_ref):
    iota_v = jax.lax.broadcasted_iota(jnp.int32, (NV, 1), 0)
    iota_s = jax.lax.broadcasted_iota(jnp.int32, (NS, 1), 0)

    def row(i, acc):
        inst_row = inst_ref[pl.ds(i * 8, 8), :].reshape(1, 1024)
        sem_row = sem_ref[pl.ds(i * 8, 8), :].reshape(1, 1024)
        surf_row = jnp.abs(geom_ref[pl.ds(i * 8, 8), :].reshape(1, 1024)) < 1.0
        a = (inst_row == iota_v).astype(jnp.float32)
        b = ((sem_row == iota_s) & surf_row).astype(jnp.float32)
        b = jnp.where(iota_s == NS - 1, 1.0, b)
        return acc + jax.lax.dot_general(
            a, b, (((1,), (1,)), ((), ())),
            preferred_element_type=jnp.float32)

    hist_ref[...] = jax.lax.fori_loop(
        0, N // 1024, row, jnp.zeros((NV, NS), jnp.float32))


def _table_kernel(ids_ref, hist_ref, lut_ref, sem_arr_ref):
    ids2d = ids_ref[...]
    hist = hist_ref[...]
    iota_col = jax.lax.broadcasted_iota(jnp.int32, (NV, 1), 0)
    keep = jnp.any(iota_col == ids2d, axis=1, keepdims=True)
    count_all = hist[:, NS - 1:NS]
    present = keep & (count_all > 0.0) & (iota_col >= 1)
    present_f = present.astype(jnp.float32)
    r_i = jax.lax.broadcasted_iota(jnp.int32, (NV, NV), 0)
    c_i = jax.lax.broadcasted_iota(jnp.int32, (NV, NV), 1)
    lt = (c_i < r_i).astype(jnp.float32)
    rank = jax.lax.dot_general(lt, present_f, (((1,), (0,)), ((), ())),
                               preferred_element_type=jnp.float32)
    covered = jnp.sum(jnp.where(keep & (iota_col >= 1), count_all, 0.0))
    present0 = (covered < float(N)).astype(jnp.int32)
    pid = rank.astype(jnp.int32) + 2 + present0
    lut_ref[...] = jnp.where(present, pid, 0)

    iota_s = jax.lax.broadcasted_iota(jnp.int32, (NV, NS), 1)
    hist_m = jnp.where(iota_s < NUM_LABELS, hist, -1.0)
    cnt = jnp.sum(jnp.where(iota_s < NUM_LABELS, hist, 0.0), axis=1,
                  keepdims=True)
    mx = jnp.max(hist_m, axis=1, keepdims=True)
    sel = jnp.min(jnp.where((hist_m == mx) & (iota_s < NUM_LABELS),
                            iota_s, NS), axis=1, keepdims=True)
    do_sem = present & (cnt > 0.0)
    iota_j = jax.lax.broadcasted_iota(jnp.int32, (1, 512), 1)
    m = ((pid == iota_j) & do_sem).astype(jnp.float32)
    sel_f = sel.astype(jnp.float32)
    sem_vals = jax.lax.dot_general(sel_f, m, (((0,), (0,)), ((), ())),
                                   preferred_element_type=jnp.float32)
    sem_vals = jnp.where(iota_j == 1, 10.0, sem_vals)
    sem_vals = jnp.where(iota_j == 2, 11.0, sem_vals)
    sem_arr_ref[...] = sem_vals


def _map_fill_kernel(ids_ref, lut_ref, inst_ref, sem_ref, geom_ref,
                     pan_ref, zpad_ref, ypad_ref, xpad_ref):
    inst = inst_ref[...]
    sem = sem_ref[...]
    surf = jnp.abs(geom_ref[...]) < 1.0

    def body(k, p):
        v = ids_ref[k]
        lv = lut_ref[v]
        return jnp.where((inst == v) & surf, lv, p)

    p = jax.lax.fori_loop(0, 64, body, jnp.zeros_like(inst))
    s_int = surf.astype(jnp.int32)
    p = jnp.where(sem == 0, 1, p)
    p = jnp.where(sem == s_int, 2, p)

    unassigned = surf & (p == 0)
    penc = jnp.where(p == 0, INF, p)

    lanes = jax.lax.broadcasted_iota(jnp.int32, (D, H * W), 1)
    wmod = jnp.bitwise_and(lanes, W - 1)
    hidx = jnp.right_shift(lanes, 6)

    zpad_ref[...] = jnp.full((D, H * W + 256), INF, jnp.int32)
    zpad_ref[:, 128:128 + H * W] = penc
    t = jnp.full((D, H * W), INF, jnp.int32)
    for dz in range(-RADIUS, RADIUS):
        s = zpad_ref[:, 128 + dz:128 + dz + H * W] + (dz + RADIUS) * 1024
        ok = (wmod + dz >= 0) & (wmod + dz < W)
        t = jnp.minimum(t, jnp.where(ok, s, INF))

    ypad_ref[...] = jnp.full((D, H * W + 512), INF, jnp.int32)
    ypad_ref[:, 256:256 + H * W] = t
    t = jnp.full((D, H * W), INF, jnp.int32)
    for dy in range(-RADIUS, RADIUS):
        s = ypad_ref[:, 256 + dy * W:256 + dy * W + H * W] + (dy + RADIUS) * 6144
        ok = (hidx + dy >= 0) & (hidx + dy < H)
        t = jnp.minimum(t, jnp.where(ok, s, INF))

    xpad_ref[...] = jnp.full((D + 8, H * W), INF, jnp.int32)
    xpad_ref[RADIUS:RADIUS + D, :] = t
    t = jnp.full((D, H * W), INF, jnp.int32)
    for dx in range(-RADIUS, RADIUS):
        s = xpad_ref[RADIUS + dx:RADIUS + dx + D, :] + (dx + RADIUS) * 36864
        t = jnp.minimum(t, s)

    fill = jnp.where(t < THRESH, jnp.bitwise_and(t, 1023), 0)
    out = jnp.where(unassigned, fill, p)
    pan_ref[...] = out.astype(jnp.float32)


@functools.partial(jax.jit, static_argnames=("interpret",))
def _run(instances2d_ids, instance3d, semantic3d_label, geometry,
         interpret=False):
    inst_r = instance3d.reshape(N // 128, 128)
    sem_r = semantic3d_label.reshape(N // 128, 128)
    geom_r = geometry.reshape(N // 128, 128)

    hist = pl.pallas_call(
        _hist_kernel,
        in_specs=[pl.BlockSpec(memory_space=pltpu.VMEM)] * 3,
        out_specs=pl.BlockSpec(memory_space=pltpu.VMEM),
        out_shape=jax.ShapeDtypeStruct((NV, NS), jnp.float32),
        interpret=interpret,
    )(inst_r, sem_r, geom_r)

    ids2d = (instances2d_ids.astype(jnp.int32) + 1).reshape(1, 64)
    lut, sem_arr = pl.pallas_call(
        _table_kernel,
        in_specs=[pl.BlockSpec(memory_space=pltpu.VMEM),
                  pl.BlockSpec(memory_space=pltpu.VMEM)],
        out_specs=[pl.BlockSpec(memory_space=pltpu.VMEM),
                   pl.BlockSpec(memory_space=pltpu.VMEM)],
        out_shape=[jax.ShapeDtypeStruct((NV, 1), jnp.int32),
                   jax.ShapeDtypeStruct((1, 512), jnp.float32)],
        interpret=interpret,
    )(ids2d, hist)

    inst2 = instance3d.reshape(D, H * W)
    sem2 = semantic3d_label.reshape(D, H * W)
    geom2 = geometry.reshape(D, H * W)
    pan = pl.pallas_call(
        _map_fill_kernel,
        in_specs=[pl.BlockSpec(memory_space=pltpu.SMEM),
                  pl.BlockSpec(memory_space=pltpu.SMEM),
                  pl.BlockSpec(memory_space=pltpu.VMEM),
                  pl.BlockSpec(memory_space=pltpu.VMEM),
                  pl.BlockSpec(memory_space=pltpu.VMEM)],
        out_specs=pl.BlockSpec(memory_space=pltpu.VMEM),
        out_shape=jax.ShapeDtypeStruct((D, H * W), jnp.float32),
        scratch_shapes=[pltpu.VMEM((D, H * W + 256), jnp.int32),
                        pltpu.VMEM((D, H * W + 512), jnp.int32),
                        pltpu.VMEM((D + 8, H * W), jnp.int32)],
        interpret=interpret,
    )(ids2d.reshape(64), lut.reshape(NV), inst2, sem2, geom2)

    return pan.reshape(1, 1, D, H, W), sem_arr.reshape(512)


def kernel(instances2d_ids, instance3d, semantic3d_label, geometry):
    return _run(instances2d_ids, instance3d, semantic3d_label, geometry)


# speedup vs baseline: 65.7241x; 1.0055x over previous
import functools

import jax
import jax.numpy as jnp
from jax.experimental import pallas as pl
from jax.experimental.pallas import tpu as pltpu

D = H = W = 64
N = D * H * W
NV = 128
NS = 32
NUM_LABELS = 20
BLK = 4096
INF = 1 << 25
THRESH = 1 << 24
RADIUS = 3


def _hist_table_kernel(inst_ref, sem_ref, geom_ref, ids_ref,
                       lut_ref, sem_arr_ref):
    iota_v = jax.lax.broadcasted_iota(jnp.int32, (NV, 1), 0)
    iota_s = jax.lax.broadcasted_iota(jnp.int32, (NS, 1), 0)

    def row(i, acc):
        inst_row = inst_ref[pl.ds(i * 8, 8), :].reshape(1, 1024)
        sem_row = sem_ref[pl.ds(i * 8, 8), :].reshape(1, 1024)
        surf_row = jnp.abs(geom_ref[pl.ds(i * 8, 8), :].reshape(1, 1024)) < 1.0
        a = (inst_row == iota_v).astype(jnp.float32)
        b = ((sem_row == iota_s) & surf_row).astype(jnp.float32)
        b = jnp.where(iota_s == NS - 1, 1.0, b)
        return acc + jax.lax.dot_general(
            a, b, (((1,), (1,)), ((), ())),
            preferred_element_type=jnp.float32)

    hist = jax.lax.fori_loop(
        0, N // 1024, row, jnp.zeros((NV, NS), jnp.float32))

    ids2d = ids_ref[...]
    iota_col = jax.lax.broadcasted_iota(jnp.int32, (NV, 1), 0)
    keep = jnp.any(iota_col == ids2d, axis=1, keepdims=True)
    count_all = hist[:, NS - 1:NS]
    present = keep & (count_all > 0.0) & (iota_col >= 1)
    present_f = present.astype(jnp.float32)
    r_i = jax.lax.broadcasted_iota(jnp.int32, (NV, NV), 0)
    c_i = jax.lax.broadcasted_iota(jnp.int32, (NV, NV), 1)
    lt = (c_i < r_i).astype(jnp.float32)
    rank = jax.lax.dot_general(lt, present_f, (((1,), (0,)), ((), ())),
                               preferred_element_type=jnp.float32)
    covered = jnp.sum(jnp.where(keep & (iota_col >= 1), count_all, 0.0))
    present0 = (covered < float(N)).astype(jnp.int32)
    pid = rank.astype(jnp.int32) + 2 + present0
    lut_ref[...] = jnp.where(present, pid, 0)

    iota_s = jax.lax.broadcasted_iota(jnp.int32, (NV, NS), 1)
    hist_m = jnp.where(iota_s < NUM_LABELS, hist, -1.0)
    cnt = jnp.sum(jnp.where(iota_s < NUM_LABELS, hist, 0.0), axis=1,
                  keepdims=True)
    mx = jnp.max(hist_m, axis=1, keepdims=True)
    sel = jnp.min(jnp.where((hist_m == mx) & (iota_s < NUM_LABELS),
                            iota_s, NS), axis=1, keepdims=True)
    do_sem = present & (cnt > 0.0)
    iota_j = jax.lax.broadcasted_iota(jnp.int32, (1, 512), 1)
    m = ((pid == iota_j) & do_sem).astype(jnp.float32)
    sel_f = sel.astype(jnp.float32)
    sem_vals = jax.lax.dot_general(sel_f, m, (((0,), (0,)), ((), ())),
                                   preferred_element_type=jnp.float32)
    sem_vals = jnp.where(iota_j == 1, 10.0, sem_vals)
    sem_vals = jnp.where(iota_j == 2, 11.0, sem_vals)
    sem_arr_ref[...] = sem_vals


def _map_fill_kernel(ids_ref, lut_ref, inst_ref, sem_ref, geom_ref,
                     pan_ref, zpad_ref, ypad_ref, xpad_ref):
    inst = inst_ref[...]
    sem = sem_ref[...]
    surf = jnp.abs(geom_ref[...]) < 1.0

    instm = jnp.where(surf, inst, -1)

    def body(k, p):
        v = ids_ref[k]
        lv = lut_ref[v]
        return jnp.where(instm == v, lv, p)

    p = jax.lax.fori_loop(0, 64, body, jnp.zeros_like(inst))
    s_int = surf.astype(jnp.int32)
    p = jnp.where(sem == 0, 1, p)
    p = jnp.where(sem == s_int, 2, p)

    unassigned = surf & (p == 0)
    penc = jnp.where(p == 0, INF, p)

    lanes = jax.lax.broadcasted_iota(jnp.int32, (D, H * W), 1)
    wmod = jnp.bitwise_and(lanes, W - 1)
    hidx = jnp.right_shift(lanes, 6)

    zpad_ref[...] = jnp.full((D, H * W + 256), INF, jnp.int32)
    zpad_ref[:, 128:128 + H * W] = penc
    t = jnp.full((D, H * W), INF, jnp.int32)
    for dz in range(-RADIUS, RADIUS):
        s = zpad_ref[:, 128 + dz:128 + dz + H * W] + (dz + RADIUS) * 1024
        ok = (wmod + dz >= 0) & (wmod + dz < W)
        t = jnp.minimum(t, jnp.where(ok, s, INF))

    ypad_ref[...] = jnp.full((D, H * W + 512), INF, jnp.int32)
    ypad_ref[:, 256:256 + H * W] = t
    t = jnp.full((D, H * W), INF, jnp.int32)
    for dy in range(-RADIUS, RADIUS):
        s = ypad_ref[:, 256 + dy * W:256 + dy * W + H * W] + (dy + RADIUS) * 6144
        ok = (hidx + dy >= 0) & (hidx + dy < H)
        t = jnp.minimum(t, jnp.where(ok, s, INF))

    xpad_ref[...] = jnp.full((D + 8, H * W), INF, jnp.int32)
    xpad_ref[RADIUS:RADIUS + D, :] = t
    t = jnp.full((D, H * W), INF, jnp.int32)
    for dx in range(-RADIUS, RADIUS):
        s = xpad_ref[RADIUS + dx:RADIUS + dx + D, :] + (dx + RADIUS) * 36864
        t = jnp.minimum(t, s)

    fill = jnp.where(t < THRESH, jnp.bitwise_and(t, 1023), 0)
    out = jnp.where(unassigned, fill, p)
    pan_ref[...] = out.astype(jnp.float32)


@functools.partial(jax.jit, static_argnames=("interpret",))
def _run(instances2d_ids, instance3d, semantic3d_label, geometry,
         interpret=False):
    inst_r = instance3d.reshape(N // 128, 128)
    sem_r = semantic3d_label.reshape(N // 128, 128)
    geom_r = geometry.reshape(N // 128, 128)

    ids2d = (instances2d_ids.astype(jnp.int32) + 1).reshape(1, 64)
    lut, sem_arr = pl.pallas_call(
        _hist_table_kernel,
        in_specs=[pl.BlockSpec(memory_space=pltpu.VMEM)] * 4,
        out_specs=[pl.BlockSpec(memory_space=pltpu.VMEM),
                   pl.BlockSpec(memory_space=pltpu.VMEM)],
        out_shape=[jax.ShapeDtypeStruct((NV, 1), jnp.int32),
                   jax.ShapeDtypeStruct((1, 512), jnp.float32)],
        interpret=interpret,
    )(inst_r, sem_r, geom_r, ids2d)

    inst2 = instance3d.reshape(D, H * W)
    sem2 = semantic3d_label.reshape(D, H * W)
    geom2 = geometry.reshape(D, H * W)
    pan = pl.pallas_call(
        _map_fill_kernel,
        in_specs=[pl.BlockSpec(memory_space=pltpu.SMEM),
                  pl.BlockSpec(memory_space=pltpu.SMEM),
                  pl.BlockSpec(memory_space=pltpu.VMEM),
                  pl.BlockSpec(memory_space=pltpu.VMEM),
                  pl.BlockSpec(memory_space=pltpu.VMEM)],
        out_specs=pl.BlockSpec(memory_space=pltpu.VMEM),
        out_shape=jax.ShapeDtypeStruct((D, H * W), jnp.float32),
        scratch_shapes=[pltpu.VMEM((D, H * W + 256), jnp.int32),
                        pltpu.VMEM((D, H * W + 512), jnp.int32),
                        pltpu.VMEM((D + 8, H * W), jnp.int32)],
        interpret=interpret,
    )(ids2d.reshape(64), lut.reshape(NV), inst2, sem2, geom2)

    return pan.reshape(1, 1, D, H, W), sem_arr.reshape(512)


def kernel(instances2d_ids, instance3d, semantic3d_label, geometry):
    return _run(instances2d_ids, instance3d, semantic3d_label, geometry)


# speedup vs baseline: 65.7573x; 1.0005x over previous
import functools

import jax
import jax.numpy as jnp
from jax.experimental import pallas as pl
from jax.experimental.pallas import tpu as pltpu

D = H = W = 64
N = D * H * W
NV = 128
NS = 32
NUM_LABELS = 20
INF = 1 << 25
THRESH = 1 << 24
RADIUS = 3


def _hist_table_kernel(inst_ref, sem_ref, geom_ref, ids_ref,
                       lut_ref, sem_arr_ref):
    iota_v = jax.lax.broadcasted_iota(jnp.int32, (NV, 1), 0)
    iota_s = jax.lax.broadcasted_iota(jnp.int32, (NS, 1), 0)

    def row(i, acc):
        inst_row = inst_ref[pl.ds(i * 8, 8), :].reshape(1, 1024)
        sem_row = sem_ref[pl.ds(i * 8, 8), :].reshape(1, 1024)
        surf_row = jnp.abs(geom_ref[pl.ds(i * 8, 8), :].reshape(1, 1024)) < 1.0
        a = (inst_row == iota_v).astype(jnp.float32)
        b = ((sem_row == iota_s) & surf_row).astype(jnp.float32)
        b = jnp.where(iota_s == NS - 1, 1.0, b)
        return acc + jax.lax.dot_general(
            a, b, (((1,), (1,)), ((), ())),
            preferred_element_type=jnp.float32)

    hist = jax.lax.fori_loop(
        0, N // 1024, row, jnp.zeros((NV, NS), jnp.float32))

    ids2d = ids_ref[...]
    iota_col = jax.lax.broadcasted_iota(jnp.int32, (NV, 1), 0)
    keep = jnp.any(iota_col == ids2d, axis=1, keepdims=True)
    count_all = hist[:, NS - 1:NS]
    present = keep & (count_all > 0.0) & (iota_col >= 1)
    present_f = present.astype(jnp.float32)
    r_i = jax.lax.broadcasted_iota(jnp.int32, (NV, NV), 0)
    c_i = jax.lax.broadcasted_iota(jnp.int32, (NV, NV), 1)
    lt = (c_i < r_i).astype(jnp.float32)
    rank = jax.lax.dot_general(lt, present_f, (((1,), (0,)), ((), ())),
                               preferred_element_type=jnp.float32)
    covered = jnp.sum(jnp.where(keep & (iota_col >= 1), count_all, 0.0))
    present0 = (covered < float(N)).astype(jnp.int32)
    pid = rank.astype(jnp.int32) + 2 + present0
    lut_ref[...] = jnp.where(present, pid, 0)

    iota_s = jax.lax.broadcasted_iota(jnp.int32, (NV, NS), 1)
    hist_m = jnp.where(iota_s < NUM_LABELS, hist, -1.0)
    cnt = jnp.sum(jnp.where(iota_s < NUM_LABELS, hist, 0.0), axis=1,
                  keepdims=True)
    mx = jnp.max(hist_m, axis=1, keepdims=True)
    sel = jnp.min(jnp.where((hist_m == mx) & (iota_s < NUM_LABELS),
                            iota_s, NS), axis=1, keepdims=True)
    do_sem = present & (cnt > 0.0)
    iota_j = jax.lax.broadcasted_iota(jnp.int32, (1, 512), 1)
    m = ((pid == iota_j) & do_sem).astype(jnp.float32)
    sel_f = sel.astype(jnp.float32)
    sem_vals = jax.lax.dot_general(sel_f, m, (((0,), (0,)), ((), ())),
                                   preferred_element_type=jnp.float32)
    sem_vals = jnp.where(iota_j == 1, 10.0, sem_vals)
    sem_vals = jnp.where(iota_j == 2, 11.0, sem_vals)
    sem_arr_ref[...] = sem_vals


def _map_fill_kernel(ids_ref, lut_ref, inst_ref, sem_ref, geom_ref,
                     pan_ref, zpad_ref, ypad_ref, xpad_ref):
    inst = inst_ref[...]
    sem = sem_ref[...]
    surf = jnp.abs(geom_ref[...]) < 1.0

    instm = jnp.where(surf, inst, -1)

    def body(k, p):
        v = ids_ref[k]
        lv = lut_ref[v]
        return jnp.where(instm == v, lv, p)

    p = jax.lax.fori_loop(0, 64, body, jnp.zeros_like(inst))
    s_int = surf.astype(jnp.int32)
    p = jnp.where(sem == 0, 1, p)
    p = jnp.where(sem == s_int, 2, p)

    unassigned = surf & (p == 0)
    penc = jnp.where(p == 0, INF, p)

    lanes = jax.lax.broadcasted_iota(jnp.int32, (D, H * W), 1)
    wmod = jnp.bitwise_and(lanes, W - 1)
    hidx = jnp.right_shift(lanes, 6)

    zpad_ref[...] = jnp.full((D, H * W + 256), INF, jnp.int32)
    zpad_ref[:, 128:128 + H * W] = penc
    t = jnp.full((D, H * W), INF, jnp.int32)
    for dz in range(-RADIUS, RADIUS):
        s = zpad_ref[:, 128 + dz:128 + dz + H * W] + (dz + RADIUS) * 1024
        ok = (wmod + dz >= 0) & (wmod + dz < W)
        t = jnp.minimum(t, jnp.where(ok, s, INF))

    ypad_ref[...] = jnp.full((D, H * W + 512), INF, jnp.int32)
    ypad_ref[:, 256:256 + H * W] = t
    t = jnp.full((D, H * W), INF, jnp.int32)
    for dy in range(-RADIUS, RADIUS):
        s = ypad_ref[:, 256 + dy * W:256 + dy * W + H * W] + (dy + RADIUS) * 6144
        ok = (hidx + dy >= 0) & (hidx + dy < H)
        t = jnp.minimum(t, jnp.where(ok, s, INF))

    xpad_ref[...] = jnp.full((D + 8, H * W), INF, jnp.int32)
    xpad_ref[RADIUS:RADIUS + D, :] = t
    t = jnp.full((D, H * W), INF, jnp.int32)
    for dx in range(-RADIUS, RADIUS):
        s = xpad_ref[RADIUS + dx:RADIUS + dx + D, :] + (dx + RADIUS) * 36864
        t = jnp.minimum(t, s)

    fill = jnp.where(t < THRESH, jnp.bitwise_and(t, 1023), 0)
    out = jnp.where(unassigned, fill, p)
    pan_ref[...] = out.astype(jnp.float32)


@functools.partial(jax.jit, static_argnames=("interpret",))
def _run(instances2d_ids, instance3d, semantic3d_label, geometry,
         interpret=False):
    inst_r = instance3d.reshape(N // 128, 128)
    sem_r = semantic3d_label.reshape(N // 128, 128)
    geom_r = geometry.reshape(N // 128, 128)

    ids2d = (instances2d_ids.astype(jnp.int32) + 1).reshape(1, 64)
    lut, sem_arr = pl.pallas_call(
        _hist_table_kernel,
        in_specs=[pl.BlockSpec(memory_space=pltpu.VMEM)] * 4,
        out_specs=[pl.BlockSpec(memory_space=pltpu.VMEM),
                   pl.BlockSpec(memory_space=pltpu.VMEM)],
        out_shape=[jax.ShapeDtypeStruct((NV, 1), jnp.int32),
                   jax.ShapeDtypeStruct((1, 512), jnp.float32)],
        interpret=interpret,
    )(inst_r, sem_r, geom_r, ids2d)

    inst2 = instance3d.reshape(D, H * W)
    sem2 = semantic3d_label.reshape(D, H * W)
    geom2 = geometry.reshape(D, H * W)
    pan = pl.pallas_call(
        _map_fill_kernel,
        in_specs=[pl.BlockSpec(memory_space=pltpu.SMEM),
                  pl.BlockSpec(memory_space=pltpu.SMEM),
                  pl.BlockSpec(memory_space=pltpu.VMEM),
                  pl.BlockSpec(memory_space=pltpu.VMEM),
                  pl.BlockSpec(memory_space=pltpu.VMEM)],
        out_specs=pl.BlockSpec(memory_space=pltpu.VMEM),
        out_shape=jax.ShapeDtypeStruct((D, H * W), jnp.float32),
        scratch_shapes=[pltpu.VMEM((D, H * W + 256), jnp.int32),
                        pltpu.VMEM((D, H * W + 512), jnp.int32),
                        pltpu.VMEM((D + 8, H * W), jnp.int32)],
        interpret=interpret,
    )(ids2d.reshape(64), lut.reshape(NV), inst2, sem2, geom2)

    return pan.reshape(1, 1, D, H, W), sem_arr.reshape(512)


def kernel(instances2d_ids, instance3d, semantic3d_label, geometry):
    return _run(instances2d_ids, instance3d, semantic3d_label, geometry)
